# baseline (speedup 1.0000x reference)
import jax
import jax.numpy as jnp
from jax import lax
from jax.experimental import pallas as pl
from jax.experimental.pallas import tpu as pltpu
from jax.experimental.pallas import tpu_sc as plsc

_H = 256
_HH = 128
_EP = 163840
_NDP = 10240
_RT = 320
_CH = 128


def _gelu(x):
    return 0.5 * x * (1.0 + lax.erf(x * 0.7071067811865476))


def _proj_body(x_ref, w_ref, b_ref, o_ref):
    acc = jnp.dot(x_ref[...], w_ref[...], preferred_element_type=jnp.float32)
    o_ref[...] = _gelu(acc + b_ref[...])


def _proj(x, w, b, rb=1000):
    n, k = x.shape
    return pl.pallas_call(
        _proj_body,
        grid=(n // rb,),
        in_specs=[pl.BlockSpec((rb, k), lambda i: (i, 0)),
                  pl.BlockSpec((k, _H), lambda i: (0, 0)),
                  pl.BlockSpec((1, _H), lambda i: (0, 0))],
        out_specs=pl.BlockSpec((rb, _H), lambda i: (i, 0)),
        out_shape=jax.ShapeDtypeStruct((n, _H), jnp.float32),
    )(x, w, b.reshape(1, _H))


def _combine_body(m_ref, z_ref, wl_ref, wr_ref, b_ref, o_ref):
    acc = jnp.dot(m_ref[...], wl_ref[...], preferred_element_type=jnp.float32)
    acc = acc + jnp.dot(z_ref[...], wr_ref[...], preferred_element_type=jnp.float32)
    o_ref[...] = _gelu(acc + b_ref[...])


def _combine(mean_pad, z, wl, wr, b, rb=1000):
    n = z.shape[0]
    return pl.pallas_call(
        _combine_body,
        grid=(n // rb,),
        in_specs=[pl.BlockSpec((rb, _H), lambda i: (i, 0)),
                  pl.BlockSpec((rb, _H), lambda i: (i, 0)),
                  pl.BlockSpec((_H, _H), lambda i: (0, 0)),
                  pl.BlockSpec((_H, _H), lambda i: (0, 0)),
                  pl.BlockSpec((1, _H), lambda i: (0, 0))],
        out_specs=pl.BlockSpec((rb, _H), lambda i: (i, 0)),
        out_shape=jax.ShapeDtypeStruct((n, _H), jnp.float32),
    )(mean_pad, z, wl, wr, b.reshape(1, _H))


def _edge_body(t_ref, s_ref, w1t_ref, w1s_ref, w1d_ref, b1_ref, w2_ref, b2_ref, o_ref):
    t = t_ref[...]
    s = s_ref[...]
    acc = jnp.dot(t, w1t_ref[...], preferred_element_type=jnp.float32)
    acc = acc + jnp.dot(s, w1s_ref[...], preferred_element_type=jnp.float32)
    acc = acc + jnp.dot(jnp.abs(t - s), w1d_ref[...], preferred_element_type=jnp.float32)
    h = _gelu(acc + b1_ref[...])
    o_ref[...] = jnp.dot(h, w2_ref[...], preferred_element_type=jnp.float32) + b2_ref[...]


def _edge_mlp(ztg, zsg, we1, be1, we2, be2, rb=640):
    n = ztg.shape[0]
    no = we2.shape[1]
    w1t, w1s, w1d = we1[:_H], we1[_H:2 * _H], we1[2 * _H:]
    return pl.pallas_call(
        _edge_body,
        grid=(n // rb,),
        in_specs=[pl.BlockSpec((rb, _H), lambda i: (i, 0)),
                  pl.BlockSpec((rb, _H), lambda i: (i, 0)),
                  pl.BlockSpec((_H, _H), lambda i: (0, 0)),
                  pl.BlockSpec((_H, _H), lambda i: (0, 0)),
                  pl.BlockSpec((_H, _H), lambda i: (0, 0)),
                  pl.BlockSpec((1, _H), lambda i: (0, 0)),
                  pl.BlockSpec((_H, no), lambda i: (0, 0)),
                  pl.BlockSpec((1, no), lambda i: (0, 0))],
        out_specs=pl.BlockSpec((rb, no), lambda i: (i, 0)),
        out_shape=jax.ShapeDtypeStruct((n, no), jnp.float32),
    )(ztg, zsg, w1t, w1s, w1d, be1.reshape(1, _H), we2, be2.reshape(1, no))


def _heads_body(z_ref, wb1_ref, bb1_ref, wb2_ref, bb2_ref,
                ws1_ref, bs1_ref, ws2_ref, bs2_ref, ob_ref, os_ref):
    z = z_ref[...]
    u = _gelu(jnp.dot(z, wb1_ref[...], preferred_element_type=jnp.float32) + bb1_ref[...])
    ob_ref[...] = jnp.sum(u * wb2_ref[...], axis=1, keepdims=True) + bb2_ref[0, 0]
    v = _gelu(jnp.dot(z, ws1_ref[...], preferred_element_type=jnp.float32) + bs1_ref[...])
    os_ref[...] = jax.nn.sigmoid(jnp.sum(v * ws2_ref[...], axis=1, keepdims=True) + bs2_ref[0, 0])


def _heads(z, wb1, bb1, wb2, bb2, ws1, bs1, ws2, bs2, rb=1000):
    n = z.shape[0]
    hh = wb1.shape[1]
    return pl.pallas_call(
        _heads_body,
        grid=(n // rb,),
        in_specs=[pl.BlockSpec((rb, _H), lambda i: (i, 0)),
                  pl.BlockSpec((_H, hh), lambda i: (0, 0)),
                  pl.BlockSpec((1, hh), lambda i: (0, 0)),
                  pl.BlockSpec((1, hh), lambda i: (0, 0)),
                  pl.BlockSpec((1, 1), lambda i: (0, 0)),
                  pl.BlockSpec((_H, hh), lambda i: (0, 0)),
                  pl.BlockSpec((1, hh), lambda i: (0, 0)),
                  pl.BlockSpec((1, hh), lambda i: (0, 0)),
                  pl.BlockSpec((1, 1), lambda i: (0, 0))],
        out_specs=[pl.BlockSpec((rb, 1), lambda i: (i, 0)),
                   pl.BlockSpec((rb, 1), lambda i: (i, 0))],
        out_shape=[jax.ShapeDtypeStruct((n, 1), jnp.float32),
                   jax.ShapeDtypeStruct((n, 1), jnp.float32)],
    )(z, wb1, bb1.reshape(1, hh), wb2.reshape(1, hh), bb2.reshape(1, 1),
      ws1, bs1.reshape(1, hh), ws2.reshape(1, hh), bs2.reshape(1, 1))


_HQ = 16
_NP = _H // 2 // _HQ


def _segmean2(set_a, set_b):
    mesh = plsc.VectorSubcoreMesh(core_axis_name="c", subcore_axis_name="s")

    def body(tab_a_hbm, src_a_hbm, dst_a_hbm, tab_b_hbm, src_b_hbm, dst_b_hbm,
             out_a_hbm, out_b_hbm,
             idx_v, dst_v, rows_v, ones_v, mean_v, inv_v, half_v, acc_sh, sem):
        cid = lax.axis_index("c")
        sid = lax.axis_index("s")

        @pl.loop(0, _CH)
        def _(i):
            for j in range(_HQ // 16):
                ones_v[i, pl.ds(16 * j, 16)] = jnp.ones((16,), jnp.float32)

        def zero_mean_v():
            @pl.loop(0, _RT)
            def _(i):
                for j in range(_HQ // 16):
                    mean_v[i, pl.ds(16 * j, 16)] = jnp.zeros((16,), jnp.float32)

        def zero_own_acc_rows():
            for k in range(2):
                r0 = sid * 640 + k * _RT
                pltpu.sync_copy(mean_v, acc_sh.at[pl.ds(r0, _RT)])

        zero_mean_v()

        for (table_hbm, src_hbm, dst_hbm, out_hbm) in (
                (tab_a_hbm, src_a_hbm, dst_a_hbm, out_a_hbm),
                (tab_b_hbm, src_b_hbm, dst_b_hbm, out_b_hbm)):

            zero_own_acc_rows()
            plsc.subcore_barrier()

            @pl.loop(0, _EP // 16 // _CH)
            def _(ci):
                base = sid * (_EP // 16) + ci * _CH
                pltpu.sync_copy(dst_hbm.at[pl.ds(base, _CH)], dst_v)
                pltpu.sync_copy(ones_v, acc_sh.at[dst_v], add=True)

            plsc.subcore_barrier()

            for k in range(2):
                r0 = sid * 640 + k * _RT
                pltpu.sync_copy(acc_sh.at[pl.ds(r0, _RT)], mean_v)

                @pl.loop(0, _RT)
                def _(i):
                    inv_v[k * _RT + i, :] = 1.0 / jnp.maximum(mean_v[i, pl.ds(0, 16)], 1.0)

            zero_mean_v()

            for p in range(_NP):
                qid = cid * _NP + p
                zero_own_acc_rows()
                plsc.subcore_barrier()

                @pl.loop(0, _EP // 16 // _CH)
                def _(ci):
                    base = sid * (_EP // 16) + ci * _CH
                    pltpu.sync_copy(src_hbm.at[pl.ds(base, _CH)], idx_v)
                    pltpu.sync_copy(dst_hbm.at[pl.ds(base, _CH)], dst_v)
                    for j in range(_CH // 16):
                        sl = pl.ds(16 * j, 16)
                        idx_v[sl] = idx_v[sl] + qid
                    pltpu.async_copy(table_hbm.at[idx_v], rows_v, sem).wait()
                    pltpu.sync_copy(rows_v, acc_sh.at[dst_v], add=True)

                plsc.subcore_barrier()

                for k in range(2):
                    r0 = sid * 640 + k * _RT
                    pltpu.sync_copy(acc_sh.at[pl.ds(r0, _RT)], mean_v)

                    @pl.loop(0, _RT)
                    def _(i):
                        inv = inv_v[k * _RT + i, :]
                        for j in range(_HQ // 16):
                            sl = pl.ds(16 * j, 16)
                            half_v[k * _RT + i, pl.ds(p * _HQ + 16 * j, 16)] = mean_v[i, sl] * inv

                zero_mean_v()

            pltpu.sync_copy(half_v, out_hbm.at[pl.ds(sid * 640, 640), pl.ds(cid * _HH, _HH)])

    f = pl.kernel(
        body,
        out_type=(jax.ShapeDtypeStruct((_NDP, _H), jnp.float32),
                  jax.ShapeDtypeStruct((_NDP, _H), jnp.float32)),
        mesh=mesh,
        compiler_params=pltpu.CompilerParams(use_tc_tiling_on_sc=False),
        scratch_types=[
            pltpu.VMEM((_CH,), jnp.int32),
            pltpu.VMEM((_CH,), jnp.int32),
            pltpu.VMEM((_CH, _HQ), jnp.float32),
            pltpu.VMEM((_CH, _HQ), jnp.float32),
            pltpu.VMEM((_RT, _HQ), jnp.float32),
            pltpu.VMEM((640, 16), jnp.float32),
            pltpu.VMEM((640, _HH), jnp.float32),
            pltpu.VMEM_SHARED((_NDP, _HQ), jnp.float32),
            pltpu.SemaphoreType.DMA,
        ],
    )
    return f(*set_a, *set_b)


def _gather2(zt, zs, e0, e1):
    mesh = plsc.VectorSubcoreMesh(core_axis_name="c", subcore_axis_name="s")
    per = _EP // 32

    def body(zt_hbm, zs_hbm, e0_hbm, e1_hbm, ot_hbm, os_hbm,
             i0_v, i1_v, rt_v, rs_v, sem0, sem1):
        cid = lax.axis_index("c")
        sid = lax.axis_index("s")
        wid = sid * 2 + cid

        @pl.loop(0, per // _CH)
        def _(ci):
            base = wid * per + ci * _CH
            pltpu.sync_copy(e0_hbm.at[pl.ds(base, _CH)], i0_v)
            pltpu.sync_copy(e1_hbm.at[pl.ds(base, _CH)], i1_v)
            c0 = pltpu.async_copy(zt_hbm.at[i0_v], rt_v, sem0)
            c1 = pltpu.async_copy(zs_hbm.at[i1_v], rs_v, sem1)
            c0.wait()
            c1.wait()
            pltpu.sync_copy(rt_v, ot_hbm.at[pl.ds(base, _CH)])
            pltpu.sync_copy(rs_v, os_hbm.at[pl.ds(base, _CH)])

    f = pl.kernel(
        body,
        out_type=(jax.ShapeDtypeStruct((_EP, _H), jnp.float32),
                  jax.ShapeDtypeStruct((_EP, _H), jnp.float32)),
        mesh=mesh,
        scratch_types=[
            pltpu.VMEM((_CH,), jnp.int32),
            pltpu.VMEM((_CH,), jnp.int32),
            pltpu.VMEM((_CH, _H), jnp.float32),
            pltpu.VMEM((_CH, _H), jnp.float32),
            pltpu.SemaphoreType.DMA,
            pltpu.SemaphoreType.DMA,
        ],
    )
    return f(zt, zs, e0, e1)


def kernel(x_transcript, x_symptom, edge_index_ts, edge_index_st, edge_label_index,
           Wt_proj, bt_proj, Ws_proj, bs_proj,
           Wl0_ts, bl0_ts, Wr0_ts, Wl0_st, bl0_st, Wr0_st,
           Wl1_ts, bl1_ts, Wr1_ts, Wl1_st, bl1_st, Wr1_st,
           We1, be1, We2, be2, Wb1, bb1, Wb2, bb2, Wsc1, bsc1, Wsc2, bsc2):
    nt = x_transcript.shape[0]
    ns = x_symptom.shape[0]
    e = edge_index_ts.shape[1]
    el = edge_label_index.shape[1]

    z_t = _proj(x_transcript, Wt_proj, bt_proj)
    z_s = _proj(x_symptom, Ws_proj, bs_proj)

    def _pad_idx(a, fill, n):
        a = a.astype(jnp.int32)
        return jnp.concatenate([a, jnp.full((_EP - n,), fill, jnp.int32)])

    src_ts = _pad_idx(16 * edge_index_ts[0], 0, e)
    dst_ts = _pad_idx(edge_index_ts[1], ns, e)
    src_st = _pad_idx(16 * edge_index_st[0], 0, e)
    dst_st = _pad_idx(edge_index_st[1], nt, e)
    e0 = _pad_idx(edge_label_index[0], 0, el)
    e1 = _pad_idx(edge_label_index[1], 0, el)

    stacked = (jnp.stack([Wl0_ts, Wl1_ts]), jnp.stack([bl0_ts, bl1_ts]),
               jnp.stack([Wr0_ts, Wr1_ts]), jnp.stack([Wl0_st, Wl1_st]),
               jnp.stack([bl0_st, bl1_st]), jnp.stack([Wr0_st, Wr1_st]))

    def _layer(carry, ws):
        zt, zs = carry
        wl_ts, bl_ts, wr_ts, wl_st, bl_st, wr_st = ws
        mean_s, mean_t = _segmean2((zt.reshape(16 * nt, _HQ), src_ts, dst_ts),
                                   (zs.reshape(16 * ns, _HQ), src_st, dst_st))
        new_s = _combine(mean_s, zs, wl_ts, wr_ts, bl_ts)
        new_t = _combine(mean_t, zt, wl_st, wr_st, bl_st)
        return (new_t, new_s), None

    (z_t, z_s), _ = lax.scan(_layer, (z_t, z_s), stacked)

    ztg, zsg = _gather2(z_t, z_s, e0, e1)
    edge_logits = _edge_mlp(ztg, zsg, We1, be1, We2, be2)[:el]
    binary_logit, score_frac = _heads(z_t, Wb1, bb1, Wb2, bb2, Wsc1, bsc1, Wsc2, bsc2)
    return (edge_logits, binary_logit[:, 0], score_frac[:, 0], z_t, z_s)


# speedup vs baseline: 2.0151x; 2.0151x over previous
import jax
import jax.numpy as jnp
from jax import lax
from jax.experimental import pallas as pl
from jax.experimental.pallas import tpu as pltpu
from jax.experimental.pallas import tpu_sc as plsc

_H = 256
_HH = 128
_EP = 163840
_NDP = 10240
_RT = 320
_CH = 128


def _gelu(x):
    return 0.5 * x * (1.0 + lax.erf(x * 0.7071067811865476))


def _proj_body(x_ref, w_ref, b_ref, o_ref):
    acc = jnp.dot(x_ref[...], w_ref[...], preferred_element_type=jnp.float32)
    o_ref[...] = _gelu(acc + b_ref[...])


def _proj(x, w, b, rb=1000):
    n, k = x.shape
    return pl.pallas_call(
        _proj_body,
        grid=(n // rb,),
        in_specs=[pl.BlockSpec((rb, k), lambda i: (i, 0)),
                  pl.BlockSpec((k, _H), lambda i: (0, 0)),
                  pl.BlockSpec((1, _H), lambda i: (0, 0))],
        out_specs=pl.BlockSpec((rb, _H), lambda i: (i, 0)),
        out_shape=jax.ShapeDtypeStruct((n, _H), jnp.float32),
    )(x, w, b.reshape(1, _H))


def _combine_body(m_ref, z_ref, wl_ref, wr_ref, b_ref, o_ref):
    acc = jnp.dot(m_ref[...], wl_ref[...], preferred_element_type=jnp.float32)
    acc = acc + jnp.dot(z_ref[...], wr_ref[...], preferred_element_type=jnp.float32)
    o_ref[...] = _gelu(acc + b_ref[...])


def _combine(mean_pad, z, wl, wr, b, rb=1000):
    n = z.shape[0]
    return pl.pallas_call(
        _combine_body,
        grid=(n // rb,),
        in_specs=[pl.BlockSpec((rb, _H), lambda i: (i, 0)),
                  pl.BlockSpec((rb, _H), lambda i: (i, 0)),
                  pl.BlockSpec((_H, _H), lambda i: (0, 0)),
                  pl.BlockSpec((_H, _H), lambda i: (0, 0)),
                  pl.BlockSpec((1, _H), lambda i: (0, 0))],
        out_specs=pl.BlockSpec((rb, _H), lambda i: (i, 0)),
        out_shape=jax.ShapeDtypeStruct((n, _H), jnp.float32),
    )(mean_pad, z, wl, wr, b.reshape(1, _H))


def _edge_body(t_ref, s_ref, w1t_ref, w1s_ref, w1d_ref, b1_ref, w2_ref, b2_ref, o_ref):
    t = t_ref[...]
    s = s_ref[...]
    acc = jnp.dot(t, w1t_ref[...], preferred_element_type=jnp.float32)
    acc = acc + jnp.dot(s, w1s_ref[...], preferred_element_type=jnp.float32)
    acc = acc + jnp.dot(jnp.abs(t - s), w1d_ref[...], preferred_element_type=jnp.float32)
    h = _gelu(acc + b1_ref[...])
    o_ref[...] = jnp.dot(h, w2_ref[...], preferred_element_type=jnp.float32) + b2_ref[...]


def _edge_mlp(ztg, zsg, we1, be1, we2, be2, rb=640):
    n = ztg.shape[0]
    no = we2.shape[1]
    w1t, w1s, w1d = we1[:_H], we1[_H:2 * _H], we1[2 * _H:]
    return pl.pallas_call(
        _edge_body,
        grid=(n // rb,),
        in_specs=[pl.BlockSpec((rb, _H), lambda i: (i, 0)),
                  pl.BlockSpec((rb, _H), lambda i: (i, 0)),
                  pl.BlockSpec((_H, _H), lambda i: (0, 0)),
                  pl.BlockSpec((_H, _H), lambda i: (0, 0)),
                  pl.BlockSpec((_H, _H), lambda i: (0, 0)),
                  pl.BlockSpec((1, _H), lambda i: (0, 0)),
                  pl.BlockSpec((_H, no), lambda i: (0, 0)),
                  pl.BlockSpec((1, no), lambda i: (0, 0))],
        out_specs=pl.BlockSpec((rb, no), lambda i: (i, 0)),
        out_shape=jax.ShapeDtypeStruct((n, no), jnp.float32),
    )(ztg, zsg, w1t, w1s, w1d, be1.reshape(1, _H), we2, be2.reshape(1, no))


def _heads_body(z_ref, wb1_ref, bb1_ref, wb2_ref, bb2_ref,
                ws1_ref, bs1_ref, ws2_ref, bs2_ref, ob_ref, os_ref):
    z = z_ref[...]
    u = _gelu(jnp.dot(z, wb1_ref[...], preferred_element_type=jnp.float32) + bb1_ref[...])
    ob_ref[...] = jnp.sum(u * wb2_ref[...], axis=1, keepdims=True) + bb2_ref[0, 0]
    v = _gelu(jnp.dot(z, ws1_ref[...], preferred_element_type=jnp.float32) + bs1_ref[...])
    os_ref[...] = jax.nn.sigmoid(jnp.sum(v * ws2_ref[...], axis=1, keepdims=True) + bs2_ref[0, 0])


def _heads(z, wb1, bb1, wb2, bb2, ws1, bs1, ws2, bs2, rb=1000):
    n = z.shape[0]
    hh = wb1.shape[1]
    return pl.pallas_call(
        _heads_body,
        grid=(n // rb,),
        in_specs=[pl.BlockSpec((rb, _H), lambda i: (i, 0)),
                  pl.BlockSpec((_H, hh), lambda i: (0, 0)),
                  pl.BlockSpec((1, hh), lambda i: (0, 0)),
                  pl.BlockSpec((1, hh), lambda i: (0, 0)),
                  pl.BlockSpec((1, 1), lambda i: (0, 0)),
                  pl.BlockSpec((_H, hh), lambda i: (0, 0)),
                  pl.BlockSpec((1, hh), lambda i: (0, 0)),
                  pl.BlockSpec((1, hh), lambda i: (0, 0)),
                  pl.BlockSpec((1, 1), lambda i: (0, 0))],
        out_specs=[pl.BlockSpec((rb, 1), lambda i: (i, 0)),
                   pl.BlockSpec((rb, 1), lambda i: (i, 0))],
        out_shape=[jax.ShapeDtypeStruct((n, 1), jnp.float32),
                   jax.ShapeDtypeStruct((n, 1), jnp.float32)],
    )(z, wb1, bb1.reshape(1, hh), wb2.reshape(1, hh), bb2.reshape(1, 1),
      ws1, bs1.reshape(1, hh), ws2.reshape(1, hh), bs2.reshape(1, 1))


_HQ = 16
_NP = _H // 2 // _HQ


def _segmean2(set_a, set_b):
    mesh = plsc.VectorSubcoreMesh(core_axis_name="c", subcore_axis_name="s")
    nring = 4
    nchunk = _EP // 16 // _CH
    nouter = nchunk // nring

    def body(tab_a_hbm, src_a_hbm, dst_a_hbm, tab_b_hbm, src_b_hbm, dst_b_hbm,
             out_a_hbm, out_b_hbm,
             src_r0, src_r1, src_r2, src_r3,
             dst_r0, dst_r1, dst_r2, dst_r3,
             idx_r0, idx_r1, idx_r2, idx_r3,
             rows_r0, rows_r1, rows_r2, rows_r3, ones_v,
             mean_v, inv_v, half_v, acc_sh,
             lsem0, lsem1, lsem2, lsem3,
             dsem0, dsem1, dsem2, dsem3,
             gsem0, gsem1, gsem2, gsem3,
             ssem0, ssem1, ssem2, ssem3):
        src_r = (src_r0, src_r1, src_r2, src_r3)
        dst_r = (dst_r0, dst_r1, dst_r2, dst_r3)
        idx_r = (idx_r0, idx_r1, idx_r2, idx_r3)
        rows_r = (rows_r0, rows_r1, rows_r2, rows_r3)
        lsem = (lsem0, lsem1, lsem2, lsem3)
        dsem = (dsem0, dsem1, dsem2, dsem3)
        gsem = (gsem0, gsem1, gsem2, gsem3)
        ssem = (ssem0, ssem1, ssem2, ssem3)
        cid = lax.axis_index("c")
        sid = lax.axis_index("s")

        @pl.loop(0, _CH)
        def _(i):
            for j in range(_HQ // 16):
                ones_v[i, pl.ds(16 * j, 16)] = jnp.ones((16,), jnp.float32)

        def zero_mean_v():
            @pl.loop(0, _RT)
            def _(i):
                for j in range(_HQ // 16):
                    mean_v[i, pl.ds(16 * j, 16)] = jnp.zeros((16,), jnp.float32)

        def zero_own_acc_rows():
            for k in range(2):
                r0 = sid * 640 + k * _RT
                pltpu.sync_copy(mean_v, acc_sh.at[pl.ds(r0, _RT)])

        zero_mean_v()

        for (table_hbm, src_hbm, dst_hbm, out_hbm) in (
                (tab_a_hbm, src_a_hbm, dst_a_hbm, out_a_hbm),
                (tab_b_hbm, src_b_hbm, dst_b_hbm, out_b_hbm)):

            zero_own_acc_rows()
            plsc.subcore_barrier()

            @pl.loop(0, nouter)
            def _(cio):
                ld = []
                for b in range(nring):
                    base = sid * (_EP // 16) + (cio * nring + b) * _CH
                    ld.append(pltpu.async_copy(
                        dst_hbm.at[pl.ds(base, _CH)], dst_r[b], dsem[b]))
                ss = []
                for b in range(nring):
                    ld[b].wait()
                    ss.append(pltpu.async_copy(
                        ones_v, acc_sh.at[dst_r[b]], ssem[b], add=True))
                for b in range(nring):
                    ss[b].wait()

            plsc.subcore_barrier()

            for k in range(2):
                r0 = sid * 640 + k * _RT
                pltpu.sync_copy(acc_sh.at[pl.ds(r0, _RT)], mean_v)

                @pl.loop(0, _RT)
                def _(i):
                    inv_v[k * _RT + i, :] = 1.0 / jnp.maximum(mean_v[i, pl.ds(0, 16)], 1.0)

            zero_mean_v()

            for p in range(_NP):
                qid = cid * _NP + p
                zero_own_acc_rows()
                plsc.subcore_barrier()

                @pl.loop(0, nouter)
                def _(cio):
                    lds, ldd = [], []
                    for b in range(nring):
                        base = sid * (_EP // 16) + (cio * nring + b) * _CH
                        lds.append(pltpu.async_copy(
                            src_hbm.at[pl.ds(base, _CH)], src_r[b], lsem[b]))
                        ldd.append(pltpu.async_copy(
                            dst_hbm.at[pl.ds(base, _CH)], dst_r[b], dsem[b]))
                    gs = []
                    for b in range(nring):
                        lds[b].wait()
                        for j in range(_CH // 16):
                            sl = pl.ds(16 * j, 16)
                            idx_r[b][sl] = src_r[b][sl] + qid
                        gs.append(pltpu.async_copy(
                            table_hbm.at[idx_r[b]], rows_r[b], gsem[b]))
                    ss = []
                    for b in range(nring):
                        gs[b].wait()
                        ldd[b].wait()
                        ss.append(pltpu.async_copy(
                            rows_r[b], acc_sh.at[dst_r[b]], ssem[b], add=True))
                    for b in range(nring):
                        ss[b].wait()

                plsc.subcore_barrier()

                for k in range(2):
                    r0 = sid * 640 + k * _RT
                    pltpu.sync_copy(acc_sh.at[pl.ds(r0, _RT)], mean_v)

                    @pl.loop(0, _RT)
                    def _(i):
                        inv = inv_v[k * _RT + i, :]
                        for j in range(_HQ // 16):
                            sl = pl.ds(16 * j, 16)
                            half_v[k * _RT + i, pl.ds(p * _HQ + 16 * j, 16)] = mean_v[i, sl] * inv

                zero_mean_v()

            pltpu.sync_copy(half_v, out_hbm.at[pl.ds(sid * 640, 640), pl.ds(cid * _HH, _HH)])

    f = pl.kernel(
        body,
        out_type=(jax.ShapeDtypeStruct((_NDP, _H), jnp.float32),
                  jax.ShapeDtypeStruct((_NDP, _H), jnp.float32)),
        mesh=mesh,
        compiler_params=pltpu.CompilerParams(use_tc_tiling_on_sc=False),
        scratch_types=(
            [pltpu.VMEM((_CH,), jnp.int32) for _ in range(12)]
            + [pltpu.VMEM((_CH, _HQ), jnp.float32) for _ in range(5)]
            + [pltpu.VMEM((_RT, _HQ), jnp.float32),
               pltpu.VMEM((640, 16), jnp.float32),
               pltpu.VMEM((640, _HH), jnp.float32),
               pltpu.VMEM_SHARED((_NDP, _HQ), jnp.float32)]
            + [pltpu.SemaphoreType.DMA for _ in range(16)]),
    )
    return f(*set_a, *set_b)


_GCH = 64


def _gather2(zt, zs, e0, e1):
    mesh = plsc.VectorSubcoreMesh(core_axis_name="c", subcore_axis_name="s")
    per = _EP // 32
    nouter = per // (2 * _GCH)

    def body(zt_hbm, zs_hbm, e0_hbm, e1_hbm, ot_hbm, os_hbm,
             i0_r0, i0_r1, i1_r0, i1_r1,
             rt_r0, rt_r1, rs_r0, rs_r1,
             l0sem0, l0sem1, l1sem0, l1sem1,
             gtsem0, gtsem1, gssem0, gssem1,
             wtsem0, wtsem1, wssem0, wssem1):
        i0_r = (i0_r0, i0_r1)
        i1_r = (i1_r0, i1_r1)
        rt_r = (rt_r0, rt_r1)
        rs_r = (rs_r0, rs_r1)
        l0sem = (l0sem0, l0sem1)
        l1sem = (l1sem0, l1sem1)
        gtsem = (gtsem0, gtsem1)
        gssem = (gssem0, gssem1)
        wtsem = (wtsem0, wtsem1)
        wssem = (wssem0, wssem1)
        cid = lax.axis_index("c")
        sid = lax.axis_index("s")
        wid = sid * 2 + cid

        @pl.loop(0, nouter)
        def _(cio):
            ld = []
            for b in range(2):
                base = wid * per + (cio * 2 + b) * _GCH
                ld.append((pltpu.async_copy(e0_hbm.at[pl.ds(base, _GCH)], i0_r[b], l0sem[b]),
                           pltpu.async_copy(e1_hbm.at[pl.ds(base, _GCH)], i1_r[b], l1sem[b])))
            gg = []
            for b in range(2):
                ld[b][0].wait()
                gt = pltpu.async_copy(zt_hbm.at[i0_r[b]], rt_r[b], gtsem[b])
                ld[b][1].wait()
                gs = pltpu.async_copy(zs_hbm.at[i1_r[b]], rs_r[b], gssem[b])
                gg.append((gt, gs))
            ww = []
            for b in range(2):
                base = wid * per + (cio * 2 + b) * _GCH
                gg[b][0].wait()
                wt = pltpu.async_copy(rt_r[b], ot_hbm.at[pl.ds(base, _GCH)], wtsem[b])
                gg[b][1].wait()
                ws = pltpu.async_copy(rs_r[b], os_hbm.at[pl.ds(base, _GCH)], wssem[b])
                ww.append((wt, ws))
            for b in range(2):
                ww[b][0].wait()
                ww[b][1].wait()

    f = pl.kernel(
        body,
        out_type=(jax.ShapeDtypeStruct((_EP, _H), jnp.float32),
                  jax.ShapeDtypeStruct((_EP, _H), jnp.float32)),
        mesh=mesh,
        scratch_types=(
            [pltpu.VMEM((_GCH,), jnp.int32) for _ in range(4)]
            + [pltpu.VMEM((_GCH, _H), jnp.float32) for _ in range(4)]
            + [pltpu.SemaphoreType.DMA for _ in range(12)]),
    )
    return f(zt, zs, e0, e1)


def kernel(x_transcript, x_symptom, edge_index_ts, edge_index_st, edge_label_index,
           Wt_proj, bt_proj, Ws_proj, bs_proj,
           Wl0_ts, bl0_ts, Wr0_ts, Wl0_st, bl0_st, Wr0_st,
           Wl1_ts, bl1_ts, Wr1_ts, Wl1_st, bl1_st, Wr1_st,
           We1, be1, We2, be2, Wb1, bb1, Wb2, bb2, Wsc1, bsc1, Wsc2, bsc2):
    nt = x_transcript.shape[0]
    ns = x_symptom.shape[0]
    e = edge_index_ts.shape[1]
    el = edge_label_index.shape[1]

    z_t = _proj(x_transcript, Wt_proj, bt_proj)
    z_s = _proj(x_symptom, Ws_proj, bs_proj)

    def _pad_idx(a, fill, n):
        a = a.astype(jnp.int32)
        return jnp.concatenate([a, jnp.full((_EP - n,), fill, jnp.int32)])

    src_ts = _pad_idx(16 * edge_index_ts[0], 0, e)
    dst_ts = _pad_idx(edge_index_ts[1], ns, e)
    src_st = _pad_idx(16 * edge_index_st[0], 0, e)
    dst_st = _pad_idx(edge_index_st[1], nt, e)
    e0 = _pad_idx(edge_label_index[0], 0, el)
    e1 = _pad_idx(edge_label_index[1], 0, el)

    stacked = (jnp.stack([Wl0_ts, Wl1_ts]), jnp.stack([bl0_ts, bl1_ts]),
               jnp.stack([Wr0_ts, Wr1_ts]), jnp.stack([Wl0_st, Wl1_st]),
               jnp.stack([bl0_st, bl1_st]), jnp.stack([Wr0_st, Wr1_st]))

    def _layer(carry, ws):
        zt, zs = carry
        wl_ts, bl_ts, wr_ts, wl_st, bl_st, wr_st = ws
        mean_s, mean_t = _segmean2((zt.reshape(16 * nt, _HQ), src_ts, dst_ts),
                                   (zs.reshape(16 * ns, _HQ), src_st, dst_st))
        new_s = _combine(mean_s, zs, wl_ts, wr_ts, bl_ts)
        new_t = _combine(mean_t, zt, wl_st, wr_st, bl_st)
        return (new_t, new_s), None

    (z_t, z_s), _ = lax.scan(_layer, (z_t, z_s), stacked)

    ztg, zsg = _gather2(z_t, z_s, e0, e1)
    edge_logits = _edge_mlp(ztg, zsg, We1, be1, We2, be2)[:el]
    binary_logit, score_frac = _heads(z_t, Wb1, bb1, Wb2, bb2, Wsc1, bsc1, Wsc2, bsc2)
    return (edge_logits, binary_logit[:, 0], score_frac[:, 0], z_t, z_s)
